# Initial kernel scaffold; baseline (speedup 1.0000x reference)
import jax
import jax.numpy as jnp
from jax.experimental import pallas as pl


def kernel(x, edge_index, batch, w_s1, b_s1, w_s2, b_s2, Wq, bq, Wk, bk, Wv, bv, g0, be0, Wo, bo, g1, be1, w_r, b_r):
    raise NotImplementedError("write your pallas kernel here")


# speedup vs baseline: 12.7465x; 12.7465x over previous
import functools

import jax
import jax.numpy as jnp
from jax import lax
from jax.experimental import pallas as pl
from jax.experimental.pallas import tpu as pltpu
from jax.experimental.pallas import tpu_sc as plsc

NHID = 128
ALPHA = 0.5
RATIO = 64
HEADS = 4
NGRAPH = 16
CH = 512


def _sc_pass1(src_h, dst_h, zeros1_h, out_ns, out_all,
              sidx, didx, vns, vall, stage, acc_ns, acc_all, sem,
              *, n_real, epw, nchunks, stripe):
    c = lax.axis_index("c")
    s = lax.axis_index("s")
    wid = s * 2 + c
    pltpu.sync_copy(zeros1_h, stage)
    pltpu.sync_copy(stage, acc_ns.at[pl.ds(s * stripe, stripe)])
    pltpu.sync_copy(stage, acc_all.at[pl.ds(s * stripe, stripe)])
    plsc.subcore_barrier()
    base0 = wid * epw

    def body(g, carry):
        base = base0 + g * 128
        pltpu.sync_copy(src_h.at[pl.ds(base, 128)], sidx)
        pltpu.sync_copy(dst_h.at[pl.ds(base, 128)], didx)
        for j in range(8):
            sv = sidx[pl.ds(j * 16, 16)]
            dv = didx[pl.ds(j * 16, 16)]
            is_real = sv < n_real
            one = jnp.ones((16,), jnp.float32)
            zero = jnp.zeros((16,), jnp.float32)
            vall[pl.ds(j * 16, 16)] = jnp.where(is_real, one, zero)
            vns[pl.ds(j * 16, 16)] = jnp.where(
                is_real & (sv != dv), one, zero)
        pltpu.sync_copy(vns, acc_ns.at[didx], add=True)
        pltpu.sync_copy(vall, acc_all.at[didx], add=True)
        return carry

    lax.fori_loop(0, nchunks, body, 0)
    plsc.subcore_barrier()
    np_tot = 16 * stripe
    pltpu.sync_copy(acc_ns.at[pl.ds(s * stripe, stripe)], stage)
    pltpu.sync_copy(stage, out_ns.at[pl.ds(c * np_tot + s * stripe, stripe)])
    pltpu.sync_copy(acc_all.at[pl.ds(s * stripe, stripe)], stage)
    pltpu.sync_copy(stage, out_all.at[pl.ds(c * np_tot + s * stripe, stripe)])


def _sc_pass2(src_h, dst_h, zk_h, zv_h, xs2_h, zeros2_h, zeros1_h,
              outk, outv, s2_o,
              sidx, didx, didx2, rows, valb, stage, stage1, acc, acc_s2, sem,
              *, epc, nchunks, stripe1, half, acc_rows):
    c = lax.axis_index("c")
    s = lax.axis_index("s")
    stripe_a = acc_rows // 16
    stripe_o = half // 16
    pltpu.sync_copy(zeros1_h, stage1)
    pltpu.sync_copy(stage1, acc_s2.at[pl.ds(s * stripe1, stripe1)])
    base0 = s * epc

    for p in range(2):
        lo = p * half
        pltpu.sync_copy(zeros2_h, stage)
        pltpu.sync_copy(stage, acc.at[pl.ds(s * stripe_a, stripe_a)])
        plsc.subcore_barrier()

        def body(g, carry):
            base = base0 + g * 128
            pltpu.sync_copy(src_h.at[pl.ds(base, 128)], sidx)
            pltpu.sync_copy(dst_h.at[pl.ds(base, 128)], didx)

            @pl.when(c == 0)
            def _():
                pltpu.async_copy(zk_h.at[sidx], rows, sem).wait()

            @pl.when(c == 1)
            def _():
                pltpu.async_copy(zv_h.at[sidx], rows, sem).wait()

            for j in range(8):
                dv = didx[pl.ds(j * 16, 16)]
                in_half = (dv >= lo) & (dv < lo + half)
                didx2[pl.ds(j * 16, 16)] = jnp.where(
                    in_half, dv - lo, half + (dv & 127))

            if p == 0:
                @pl.when(c == 0)
                def _():
                    pltpu.async_copy(xs2_h.at[sidx], valb, sem).wait()
                    for j in range(8):
                        sv = sidx[pl.ds(j * 16, 16)]
                        dv = didx[pl.ds(j * 16, 16)]
                        g16 = valb[pl.ds(j * 16, 16)]
                        valb[pl.ds(j * 16, 16)] = jnp.where(
                            sv != dv, g16, jnp.zeros((16,), jnp.float32))
                    pltpu.sync_copy(valb, acc_s2.at[didx], add=True)

            pltpu.sync_copy(rows, acc.at[didx2], add=True)
            return carry

        lax.fori_loop(0, nchunks, body, 0)
        plsc.subcore_barrier()
        pltpu.sync_copy(acc.at[pl.ds(s * stripe_o, stripe_o)],
                        stage.at[pl.ds(0, stripe_o)])

        @pl.when(c == 0)
        def _():
            pltpu.sync_copy(stage.at[pl.ds(0, stripe_o)],
                            outk.at[pl.ds(lo + s * stripe_o, stripe_o)])

        @pl.when(c == 1)
        def _():
            pltpu.sync_copy(stage.at[pl.ds(0, stripe_o)],
                            outv.at[pl.ds(lo + s * stripe_o, stripe_o)])

        if p == 0:
            @pl.when(c == 0)
            def _():
                pltpu.sync_copy(acc_s2.at[pl.ds(s * stripe1, stripe1)],
                                stage1)
                pltpu.sync_copy(stage1,
                                s2_o.at[pl.ds(s * stripe1, stripe1)])
        plsc.subcore_barrier()


def _k3_body(x_ref, w_ref, ns0, ns1, al0, al1,
             xw_o, xk_o, xv_o, xs2_o, dl_o, dn_o):
    xw = jnp.dot(x_ref[...], w_ref[...], preferred_element_type=jnp.float32)
    xw_o[...] = xw
    dns = ns0[...] + ns1[...]
    dal = al0[...] + al1[...]
    dinv_ns = jnp.where(dns > 0, lax.rsqrt(jnp.where(dns > 0, dns, 1.0)), 0.0)
    dinv_l = lax.rsqrt(dal + 1.0)
    xk_o[...] = dinv_l * xw[:, 0:128]
    xv_o[...] = dinv_l * xw[:, 128:256]
    xs2_o[...] = dinv_ns * xw[:, 256:257]
    dl_o[...] = dinv_l
    dn_o[...] = dinv_ns


def _k3b_body(batch_ref, ptr_o):
    blk = batch_ref[...]
    lane = lax.broadcasted_iota(jnp.int32, (1, 128), 1)
    row = jnp.zeros((1, 128), jnp.int32)
    for b in range(NGRAPH + 1):
        cnt = jnp.sum((blk < b).astype(jnp.int32))
        row = jnp.where(lane == b, cnt, row)
    ptr_o[...] = row


def _k45_body(x_ref, xw_ref, ak_ref, av_ref, as2_ref, dl_ref, dn_ref,
              bk_ref, bv_ref, bs_ref,
              kf_o, vf_o, xp_o):
    dl = dl_ref[...]
    dn = dn_ref[...]
    xw = xw_ref[...]
    dl2 = dl * dl
    kf_o[...] = dl * ak_ref[...] + dl2 * xw[:, 0:128] + bk_ref[...]
    vf_o[...] = dl * av_ref[...] + dl2 * xw[:, 128:256] + bv_ref[...]
    s2 = dn * as2_ref[...]
    score = ALPHA * xw[:, 257:258] + (1.0 - ALPHA) * s2 + bs_ref[0, 0]
    xp_o[...] = x_ref[...] * jnp.tanh(score)


def _k5_body(ptr_ref, xp_ref, k_ref, v_ref, wq_ref, bq_ref, wo_ref, bo_ref,
             g0_ref, be0_ref, g1_ref, be1_ref, wr_ref, br_ref, out_ref):
    b = pl.program_id(0)
    p0 = ptr_ref[b]
    cnt = ptr_ref[b + 1] - p0
    rows = xp_ref[pl.ds(p0, RATIO), :]
    r_iota = lax.broadcasted_iota(jnp.int32, (RATIO, 1), 0)
    qd = jnp.where(r_iota < cnt, rows, 0.0)
    q = jnp.dot(qd, wq_ref[...], preferred_element_type=jnp.float32) + bq_ref[...]
    scale = 1.0 / jnp.sqrt(jnp.asarray(float(NHID), jnp.float32))
    hd = NHID // HEADS
    nch = (cnt + CH - 1) // CH
    heads = []
    for h in range(HEADS):
        qh = q[:, h * hd:(h + 1) * hd]

        def body(j, carry):
            m, l, acc = carry
            base = p0 + j * CH
            kc = k_ref[pl.ds(base, CH), :][:, h * hd:(h + 1) * hd]
            vc = v_ref[pl.ds(base, CH), :][:, h * hd:(h + 1) * hd]
            sA = lax.dot_general(qh, kc, (((1,), (1,)), ((), ())),
                                 preferred_element_type=jnp.float32) * scale
            col = lax.broadcasted_iota(jnp.int32, (RATIO, CH), 1)
            sA = sA + jnp.where(col < (cnt - j * CH), 0.0, -1e9)
            m_new = jnp.maximum(m, jnp.max(sA, axis=1, keepdims=True))
            p = jnp.exp(sA - m_new)
            corr = jnp.exp(m - m_new)
            l_new = l * corr + jnp.sum(p, axis=1, keepdims=True)
            acc_new = acc * corr + jnp.dot(p, vc,
                                           preferred_element_type=jnp.float32)
            return m_new, l_new, acc_new

        m0 = jnp.full((RATIO, 1), -1e30, jnp.float32)
        l0 = jnp.zeros((RATIO, 1), jnp.float32)
        a0 = jnp.zeros((RATIO, hd), jnp.float32)
        m, l, acc = lax.fori_loop(0, nch, body, (m0, l0, a0))
        heads.append(jnp.where(l > 0, acc / jnp.where(l > 0, l, 1.0), 0.0))
    o = q + jnp.concatenate(heads, axis=1)

    def ln(t, g, be):
        mu = jnp.mean(t, axis=-1, keepdims=True)
        var = jnp.mean((t - mu) ** 2, axis=-1, keepdims=True)
        return (t - mu) * lax.rsqrt(var + 1e-5) * g + be

    o = ln(o, g0_ref[...], be0_ref[...])
    o = o + jax.nn.relu(jnp.dot(o, wo_ref[...],
                                preferred_element_type=jnp.float32) + bo_ref[...])
    o = ln(o, g1_ref[...], be1_ref[...])
    out_ref[0] = jnp.dot(wr_ref[...], o,
                         preferred_element_type=jnp.float32) + br_ref[...]


def kernel(x, edge_index, batch, w_s1, b_s1, w_s2, b_s2, Wq, bq, Wk, bk,
           Wv, bv, g0, be0, Wo, bo, g1, be1, w_r, b_r):
    N, D = x.shape
    E = edge_index.shape[1]
    B = NGRAPH
    NP = ((N + 511 + 255) // 256) * 256
    stripe = NP // 16
    NPE = ((E + 4095) // 4096) * 4096
    epw1 = NPE // 32
    nch1 = epw1 // 128
    epc2 = NPE // 16
    nch2 = epc2 // 128
    NB = ((N + 1023) // 1024) * 1024

    f32 = jnp.float32
    x_p = jnp.zeros((NP, D), f32).at[:N].set(x)
    Wcat = jnp.concatenate(
        [Wk, Wv, w_s2, w_s1, jnp.zeros((D, 126), f32)], axis=1)
    npad = NPE - E
    src_p = jnp.concatenate(
        [edge_index[0],
         N + (jnp.arange(npad, dtype=jnp.int32) % (NP - N))])
    dst_p = jnp.concatenate(
        [edge_index[1], jnp.arange(npad, dtype=jnp.int32) % N])
    batch_rs = jnp.concatenate(
        [batch, jnp.full((NB - N,), B, jnp.int32)]).reshape(NB // 128, 128)
    zeros1 = jnp.zeros((stripe,), f32)

    mesh = plsc.VectorSubcoreMesh(core_axis_name="c", subcore_axis_name="s")
    pass1 = functools.partial(
        pl.kernel,
        out_type=[jax.ShapeDtypeStruct((2 * NP,), f32),
                  jax.ShapeDtypeStruct((2 * NP,), f32)],
        mesh=mesh,
        scratch_types=[
            pltpu.VMEM((128,), jnp.int32),
            pltpu.VMEM((128,), jnp.int32),
            pltpu.VMEM((128,), f32),
            pltpu.VMEM((128,), f32),
            pltpu.VMEM((stripe,), f32),
            pltpu.VMEM_SHARED((NP,), f32),
            pltpu.VMEM_SHARED((NP,), f32),
            pltpu.SemaphoreType.DMA,
        ])(functools.partial(_sc_pass1, n_real=N, epw=epw1,
                             nchunks=nch1, stripe=stripe))
    deg_ns_f, deg_all_f = pass1(src_p, dst_p, zeros1)
    deg_ns_p = deg_ns_f.reshape(2, NP)
    deg_all_p = deg_all_f.reshape(2, NP)

    nblk = NP // 128
    col = lambda a: a.reshape(NP, 1)
    k3 = pl.pallas_call(
        _k3_body,
        grid=(nblk,),
        in_specs=[
            pl.BlockSpec((128, 128), lambda i: (i, 0)),
            pl.BlockSpec((128, 384), lambda i: (0, 0)),
            pl.BlockSpec((128, 1), lambda i: (i, 0)),
            pl.BlockSpec((128, 1), lambda i: (i, 0)),
            pl.BlockSpec((128, 1), lambda i: (i, 0)),
            pl.BlockSpec((128, 1), lambda i: (i, 0)),
        ],
        out_specs=[
            pl.BlockSpec((128, 384), lambda i: (i, 0)),
            pl.BlockSpec((128, 128), lambda i: (i, 0)),
            pl.BlockSpec((128, 128), lambda i: (i, 0)),
            pl.BlockSpec((128, 1), lambda i: (i, 0)),
            pl.BlockSpec((128, 1), lambda i: (i, 0)),
            pl.BlockSpec((128, 1), lambda i: (i, 0)),
        ],
        out_shape=[
            jax.ShapeDtypeStruct((NP, 384), f32),
            jax.ShapeDtypeStruct((NP, 128), f32),
            jax.ShapeDtypeStruct((NP, 128), f32),
            jax.ShapeDtypeStruct((NP, 1), f32),
            jax.ShapeDtypeStruct((NP, 1), f32),
            jax.ShapeDtypeStruct((NP, 1), f32),
        ])
    XW, Xk_s, Xv_s, xs2_s, dinv_l, dinv_ns = k3(
        x_p, Wcat, col(deg_ns_p[0]), col(deg_ns_p[1]),
        col(deg_all_p[0]), col(deg_all_p[1]))

    k3b = pl.pallas_call(
        _k3b_body,
        grid=(1,),
        in_specs=[pl.BlockSpec((NB // 128, 128), lambda i: (0, 0))],
        out_specs=pl.BlockSpec((1, 128), lambda i: (0, 0)),
        out_shape=jax.ShapeDtypeStruct((1, 128), jnp.int32))
    ptr_row = k3b(batch_rs)
    ptr_pad = jnp.concatenate(
        [ptr_row[0, :B + 1], jnp.zeros((15,), jnp.int32)])

    xs2_flat = xs2_s.reshape(NP)
    epc1 = NPE // 16
    nchp = epc1 // 128
    half = NP // 2
    acc_rows = half + 128
    zeros2 = jnp.zeros((acc_rows // 16, 128), f32)
    pass2 = functools.partial(
        pl.kernel,
        out_type=[jax.ShapeDtypeStruct((NP, 128), f32),
                  jax.ShapeDtypeStruct((NP, 128), f32),
                  jax.ShapeDtypeStruct((NP,), f32)],
        mesh=mesh,
        scratch_types=[
            pltpu.VMEM((128,), jnp.int32),
            pltpu.VMEM((128,), jnp.int32),
            pltpu.VMEM((128,), jnp.int32),
            pltpu.VMEM((128, 128), f32),
            pltpu.VMEM((128,), f32),
            pltpu.VMEM((acc_rows // 16, 128), f32),
            pltpu.VMEM((stripe,), f32),
            pltpu.VMEM_SHARED((acc_rows, 128), f32),
            pltpu.VMEM_SHARED((NP,), f32),
            pltpu.SemaphoreType.DMA,
        ])(functools.partial(_sc_pass2, epc=epc1, nchunks=nchp,
                             stripe1=stripe, half=half, acc_rows=acc_rows))
    accK, accV, acc_s2 = pass2(src_p, dst_p, Xk_s, Xv_s, xs2_flat,
                               zeros2, zeros1)

    row = lambda a: a.reshape(1, -1)
    bs = ((b_s1[0] * ALPHA + b_s2[0] * (1.0 - ALPHA))
          .reshape(1, 1).astype(f32))
    k45 = pl.pallas_call(
        _k45_body,
        grid=(nblk,),
        in_specs=[
            pl.BlockSpec((128, 128), lambda i: (i, 0)),
            pl.BlockSpec((128, 384), lambda i: (i, 0)),
            pl.BlockSpec((128, 128), lambda i: (i, 0)),
            pl.BlockSpec((128, 128), lambda i: (i, 0)),
            pl.BlockSpec((128, 1), lambda i: (i, 0)),
            pl.BlockSpec((128, 1), lambda i: (i, 0)),
            pl.BlockSpec((128, 1), lambda i: (i, 0)),
            pl.BlockSpec((1, 128), lambda i: (0, 0)),
            pl.BlockSpec((1, 128), lambda i: (0, 0)),
            pl.BlockSpec((1, 1), lambda i: (0, 0)),
        ],
        out_specs=[
            pl.BlockSpec((128, 128), lambda i: (i, 0)),
            pl.BlockSpec((128, 128), lambda i: (i, 0)),
            pl.BlockSpec((128, 128), lambda i: (i, 0)),
        ],
        out_shape=[
            jax.ShapeDtypeStruct((NP, 128), f32),
            jax.ShapeDtypeStruct((NP, 128), f32),
            jax.ShapeDtypeStruct((NP, 128), f32),
        ])
    Kfin, Vfin, xp = k45(x_p, XW, accK, accV, col(acc_s2), dinv_l, dinv_ns,
                         row(bk), row(bv), bs)

    k5 = pl.pallas_call(
        _k5_body,
        grid=(B,),
        in_specs=[
            pl.BlockSpec(memory_space=pltpu.SMEM),
            pl.BlockSpec((NP, 128), lambda b: (0, 0)),
            pl.BlockSpec((NP, 128), lambda b: (0, 0)),
            pl.BlockSpec((NP, 128), lambda b: (0, 0)),
            pl.BlockSpec((128, 128), lambda b: (0, 0)),
            pl.BlockSpec((1, 128), lambda b: (0, 0)),
            pl.BlockSpec((128, 128), lambda b: (0, 0)),
            pl.BlockSpec((1, 128), lambda b: (0, 0)),
            pl.BlockSpec((1, 128), lambda b: (0, 0)),
            pl.BlockSpec((1, 128), lambda b: (0, 0)),
            pl.BlockSpec((1, 128), lambda b: (0, 0)),
            pl.BlockSpec((1, 128), lambda b: (0, 0)),
            pl.BlockSpec((1, 64), lambda b: (0, 0)),
            pl.BlockSpec((1, 128), lambda b: (0, 0)),
        ],
        out_specs=pl.BlockSpec((1, 1, 128), lambda b: (b, 0, 0)),
        out_shape=jax.ShapeDtypeStruct((B, 1, 128), f32))
    out = k5(ptr_pad, xp, Kfin, Vfin, Wq, row(bq), Wo, row(bo),
             row(g0), row(be0), row(g1), row(be1),
             w_r.reshape(1, RATIO), jnp.broadcast_to(b_r, (1, 128)))
    return out.reshape(B, 128)


# speedup vs baseline: 24.2957x; 1.9061x over previous
import functools

import jax
import jax.numpy as jnp
from jax import lax
from jax.experimental import pallas as pl
from jax.experimental.pallas import tpu as pltpu
from jax.experimental.pallas import tpu_sc as plsc

NHID = 128
ALPHA = 0.5
RATIO = 64
HEADS = 4
NGRAPH = 16
CH = 512


def _sc_pass1(src_h, dst_h, zeros1_h, out_ns, out_all,
              sidx, didx, vns, vall, stage, acc_ns, acc_all, sem,
              *, n_real, epw, nchunks, stripe):
    c = lax.axis_index("c")
    s = lax.axis_index("s")
    wid = s * 2 + c
    pltpu.sync_copy(zeros1_h, stage)
    pltpu.sync_copy(stage, acc_ns.at[pl.ds(s * stripe, stripe)])
    pltpu.sync_copy(stage, acc_all.at[pl.ds(s * stripe, stripe)])
    plsc.subcore_barrier()
    base0 = wid * epw

    def body(g, carry):
        base = base0 + g * 128
        pltpu.sync_copy(src_h.at[pl.ds(base, 128)], sidx)
        pltpu.sync_copy(dst_h.at[pl.ds(base, 128)], didx)
        for j in range(8):
            sv = sidx[pl.ds(j * 16, 16)]
            dv = didx[pl.ds(j * 16, 16)]
            is_real = sv < n_real
            one = jnp.ones((16,), jnp.float32)
            zero = jnp.zeros((16,), jnp.float32)
            vall[pl.ds(j * 16, 16)] = jnp.where(is_real, one, zero)
            vns[pl.ds(j * 16, 16)] = jnp.where(
                is_real & (sv != dv), one, zero)
        pltpu.sync_copy(vns, acc_ns.at[didx], add=True)
        pltpu.sync_copy(vall, acc_all.at[didx], add=True)
        return carry

    lax.fori_loop(0, nchunks, body, 0)
    plsc.subcore_barrier()
    np_tot = 16 * stripe
    pltpu.sync_copy(acc_ns.at[pl.ds(s * stripe, stripe)], stage)
    pltpu.sync_copy(stage, out_ns.at[pl.ds(c * np_tot + s * stripe, stripe)])
    pltpu.sync_copy(acc_all.at[pl.ds(s * stripe, stripe)], stage)
    pltpu.sync_copy(stage, out_all.at[pl.ds(c * np_tot + s * stripe, stripe)])


def _sc_pass2(src_h, dst_h, zk_h, zv_h, xs2_h, zeros2_h, zeros1_h,
              outk, outv, s2_o,
              sidx0, sidx1, didx0, didx1, didx20, didx21, didx30, didx31,
              rows0, rows1, valb0, valb1, stage, stage1, acc, acc_s2,
              gsem0, gsem1, isem0, isem1, ssem0, ssem1,
              xsem0, xsem1, s2sem0, s2sem1,
              *, epc, nchunks, stripe1, half, acc_rows):
    c = lax.axis_index("c")
    s = lax.axis_index("s")
    n = nchunks
    stripe_a = acc_rows // 16
    stripe_o = half // 16
    pltpu.sync_copy(zeros1_h, stage1)
    pltpu.sync_copy(stage1, acc_s2.at[pl.ds(s * stripe1, stripe1)])
    base0 = s * epc
    slot = [
        (sidx0, didx0, didx20, didx30, rows0, valb0,
         gsem0, isem0, ssem0, xsem0, s2sem0),
        (sidx1, didx1, didx21, didx31, rows1, valb1,
         gsem1, isem1, ssem1, xsem1, s2sem1),
    ]

    def issue_gather(sidx, rows, gsem, valb, xsem, do_s2):
        @pl.when(c == 0)
        def _():
            pltpu.async_copy(zk_h.at[sidx], rows, gsem)
            if do_s2:
                pltpu.async_copy(xs2_h.at[sidx], valb, xsem)

        @pl.when(c == 1)
        def _():
            pltpu.async_copy(zv_h.at[sidx], rows, gsem)

    def wait_gather(sidx, rows, gsem, valb, xsem, do_s2):
        @pl.when(c == 0)
        def _():
            pltpu.make_async_copy(zk_h.at[sidx], rows, gsem).wait()
            if do_s2:
                pltpu.make_async_copy(xs2_h.at[sidx], valb, xsem).wait()

        @pl.when(c == 1)
        def _():
            pltpu.make_async_copy(zv_h.at[sidx], rows, gsem).wait()

    for p in range(2):
        lo = p * half
        do_s2 = (p == 0)
        pltpu.sync_copy(zeros2_h, stage)
        pltpu.sync_copy(stage, acc.at[pl.ds(s * stripe_a, stripe_a)])
        plsc.subcore_barrier()

        pltpu.sync_copy(src_h.at[pl.ds(base0, 128)], sidx0)
        pltpu.sync_copy(dst_h.at[pl.ds(base0, 128)], didx0)
        issue_gather(sidx0, rows0, gsem0, valb0, xsem0, do_s2)
        pltpu.async_copy(src_h.at[pl.ds(base0 + 128, 128)], sidx1, isem1)
        pltpu.async_copy(dst_h.at[pl.ds(base0 + 128, 128)], didx1, isem1)

        def substep(g, b):
            (sidxb, didxb, didx2b, didx3b, rowsb, valbb,
             gsemb, isemb, ssemb, xsemb, s2semb) = slot[b]
            (sidxo, didxo, didx2o, didx3o, rowso, valbo,
             gsemo, isemo, ssemo, xsemo, s2semo) = slot[1 - b]

            @pl.when(g >= 1)
            def _():
                pltpu.make_async_copy(rowso, acc.at[didx2o], ssemo).wait()
                if do_s2:
                    @pl.when(c == 0)
                    def _():
                        pltpu.make_async_copy(
                            valbo, acc_s2.at[didx3o], s2semo).wait()

            @pl.when(g + 1 < n)
            def _():
                base1 = base0 + (g + 1) * 128
                pltpu.make_async_copy(
                    src_h.at[pl.ds(base1, 128)], sidxo, isemo).wait()
                pltpu.make_async_copy(
                    dst_h.at[pl.ds(base1, 128)], didxo, isemo).wait()
                issue_gather(sidxo, rowso, gsemo, valbo, xsemo, do_s2)

            wait_gather(sidxb, rowsb, gsemb, valbb, xsemb, do_s2)
            for j in range(8):
                dv = didxb[pl.ds(j * 16, 16)]
                in_half = (dv >= lo) & (dv < lo + half)
                didx2b[pl.ds(j * 16, 16)] = jnp.where(
                    in_half, dv - lo, half + (dv & 127))
            if do_s2:
                @pl.when(c == 0)
                def _():
                    for j in range(8):
                        sv = sidxb[pl.ds(j * 16, 16)]
                        dv = didxb[pl.ds(j * 16, 16)]
                        g16 = valbb[pl.ds(j * 16, 16)]
                        valbb[pl.ds(j * 16, 16)] = jnp.where(
                            sv != dv, g16, jnp.zeros((16,), jnp.float32))
                        didx3b[pl.ds(j * 16, 16)] = dv

            @pl.when(g + 2 < n)
            def _():
                base2 = base0 + (g + 2) * 128
                pltpu.async_copy(src_h.at[pl.ds(base2, 128)], sidxb, isemb)
                pltpu.async_copy(dst_h.at[pl.ds(base2, 128)], didxb, isemb)

            pltpu.async_copy(rowsb, acc.at[didx2b], ssemb, add=True)
            if do_s2:
                @pl.when(c == 0)
                def _():
                    pltpu.async_copy(valbb, acc_s2.at[didx3b], s2semb,
                                     add=True)

        def body(t, carry):
            substep(2 * t, 0)
            substep(2 * t + 1, 1)
            return carry

        lax.fori_loop(0, n // 2, body, 0)
        pltpu.make_async_copy(rows1, acc.at[didx21], ssem1).wait()
        if do_s2:
            @pl.when(c == 0)
            def _():
                pltpu.make_async_copy(valb1, acc_s2.at[didx31],
                                      s2sem1).wait()
        plsc.subcore_barrier()
        pltpu.sync_copy(acc.at[pl.ds(s * stripe_o, stripe_o)],
                        stage.at[pl.ds(0, stripe_o)])

        @pl.when(c == 0)
        def _():
            pltpu.sync_copy(stage.at[pl.ds(0, stripe_o)],
                            outk.at[pl.ds(lo + s * stripe_o, stripe_o)])

        @pl.when(c == 1)
        def _():
            pltpu.sync_copy(stage.at[pl.ds(0, stripe_o)],
                            outv.at[pl.ds(lo + s * stripe_o, stripe_o)])

        if p == 0:
            @pl.when(c == 0)
            def _():
                pltpu.sync_copy(acc_s2.at[pl.ds(s * stripe1, stripe1)],
                                stage1)
                pltpu.sync_copy(stage1,
                                s2_o.at[pl.ds(s * stripe1, stripe1)])
        plsc.subcore_barrier()


def _k3_body(x_ref, w_ref, ns0, ns1, al0, al1,
             xw_o, xk_o, xv_o, xs2_o, dl_o, dn_o):
    xw = jnp.dot(x_ref[...], w_ref[...], preferred_element_type=jnp.float32)
    xw_o[...] = xw
    dns = ns0[...] + ns1[...]
    dal = al0[...] + al1[...]
    dinv_ns = jnp.where(dns > 0, lax.rsqrt(jnp.where(dns > 0, dns, 1.0)), 0.0)
    dinv_l = lax.rsqrt(dal + 1.0)
    xk_o[...] = dinv_l * xw[:, 0:128]
    xv_o[...] = dinv_l * xw[:, 128:256]
    xs2_o[...] = dinv_ns * xw[:, 256:257]
    dl_o[...] = dinv_l
    dn_o[...] = dinv_ns


def _k3b_body(batch_ref, ptr_o):
    blk = batch_ref[...]
    lane = lax.broadcasted_iota(jnp.int32, (1, 128), 1)
    row = jnp.zeros((1, 128), jnp.int32)
    for b in range(NGRAPH + 1):
        cnt = jnp.sum((blk < b).astype(jnp.int32))
        row = jnp.where(lane == b, cnt, row)
    ptr_o[...] = row


def _k45_body(x_ref, xw_ref, ak_ref, av_ref, as2_ref, dl_ref, dn_ref,
              bk_ref, bv_ref, bs_ref,
              kf_o, vf_o, xp_o):
    dl = dl_ref[...]
    dn = dn_ref[...]
    xw = xw_ref[...]
    dl2 = dl * dl
    kf_o[...] = dl * ak_ref[...] + dl2 * xw[:, 0:128] + bk_ref[...]
    vf_o[...] = dl * av_ref[...] + dl2 * xw[:, 128:256] + bv_ref[...]
    s2 = dn * as2_ref[...]
    score = ALPHA * xw[:, 257:258] + (1.0 - ALPHA) * s2 + bs_ref[0, 0]
    xp_o[...] = x_ref[...] * jnp.tanh(score)


def _k5_body(ptr_ref, xp_ref, k_ref, v_ref, wq_ref, bq_ref, wo_ref, bo_ref,
             g0_ref, be0_ref, g1_ref, be1_ref, wr_ref, br_ref, out_ref):
    b = pl.program_id(0)
    p0 = ptr_ref[b]
    cnt = ptr_ref[b + 1] - p0
    rows = xp_ref[pl.ds(p0, RATIO), :]
    r_iota = lax.broadcasted_iota(jnp.int32, (RATIO, 1), 0)
    qd = jnp.where(r_iota < cnt, rows, 0.0)
    q = jnp.dot(qd, wq_ref[...], preferred_element_type=jnp.float32) + bq_ref[...]
    scale = 1.0 / jnp.sqrt(jnp.asarray(float(NHID), jnp.float32))
    hd = NHID // HEADS
    nch = (cnt + CH - 1) // CH
    heads = []
    for h in range(HEADS):
        qh = q[:, h * hd:(h + 1) * hd]

        def body(j, carry):
            m, l, acc = carry
            base = p0 + j * CH
            kc = k_ref[pl.ds(base, CH), :][:, h * hd:(h + 1) * hd]
            vc = v_ref[pl.ds(base, CH), :][:, h * hd:(h + 1) * hd]
            sA = lax.dot_general(qh, kc, (((1,), (1,)), ((), ())),
                                 preferred_element_type=jnp.float32) * scale
            col = lax.broadcasted_iota(jnp.int32, (RATIO, CH), 1)
            sA = sA + jnp.where(col < (cnt - j * CH), 0.0, -1e9)
            m_new = jnp.maximum(m, jnp.max(sA, axis=1, keepdims=True))
            p = jnp.exp(sA - m_new)
            corr = jnp.exp(m - m_new)
            l_new = l * corr + jnp.sum(p, axis=1, keepdims=True)
            acc_new = acc * corr + jnp.dot(p, vc,
                                           preferred_element_type=jnp.float32)
            return m_new, l_new, acc_new

        m0 = jnp.full((RATIO, 1), -1e30, jnp.float32)
        l0 = jnp.zeros((RATIO, 1), jnp.float32)
        a0 = jnp.zeros((RATIO, hd), jnp.float32)
        m, l, acc = lax.fori_loop(0, nch, body, (m0, l0, a0))
        heads.append(jnp.where(l > 0, acc / jnp.where(l > 0, l, 1.0), 0.0))
    o = q + jnp.concatenate(heads, axis=1)

    def ln(t, g, be):
        mu = jnp.mean(t, axis=-1, keepdims=True)
        var = jnp.mean((t - mu) ** 2, axis=-1, keepdims=True)
        return (t - mu) * lax.rsqrt(var + 1e-5) * g + be

    o = ln(o, g0_ref[...], be0_ref[...])
    o = o + jax.nn.relu(jnp.dot(o, wo_ref[...],
                                preferred_element_type=jnp.float32) + bo_ref[...])
    o = ln(o, g1_ref[...], be1_ref[...])
    out_ref[0] = jnp.dot(wr_ref[...], o,
                         preferred_element_type=jnp.float32) + br_ref[...]


def kernel(x, edge_index, batch, w_s1, b_s1, w_s2, b_s2, Wq, bq, Wk, bk,
           Wv, bv, g0, be0, Wo, bo, g1, be1, w_r, b_r):
    N, D = x.shape
    E = edge_index.shape[1]
    B = NGRAPH
    NP = ((N + 511 + 255) // 256) * 256
    stripe = NP // 16
    NPE = ((E + 4095) // 4096) * 4096
    epw1 = NPE // 32
    nch1 = epw1 // 128
    epc2 = NPE // 16
    nch2 = epc2 // 128
    NB = ((N + 1023) // 1024) * 1024

    f32 = jnp.float32
    x_p = jnp.zeros((NP, D), f32).at[:N].set(x)
    Wcat = jnp.concatenate(
        [Wk, Wv, w_s2, w_s1, jnp.zeros((D, 126), f32)], axis=1)
    npad = NPE - E
    src_p = jnp.concatenate(
        [edge_index[0],
         N + (jnp.arange(npad, dtype=jnp.int32) % (NP - N))])
    dst_p = jnp.concatenate(
        [edge_index[1], jnp.arange(npad, dtype=jnp.int32) % N])
    batch_rs = jnp.concatenate(
        [batch, jnp.full((NB - N,), B, jnp.int32)]).reshape(NB // 128, 128)
    zeros1 = jnp.zeros((stripe,), f32)

    mesh = plsc.VectorSubcoreMesh(core_axis_name="c", subcore_axis_name="s")
    pass1 = functools.partial(
        pl.kernel,
        out_type=[jax.ShapeDtypeStruct((2 * NP,), f32),
                  jax.ShapeDtypeStruct((2 * NP,), f32)],
        mesh=mesh,
        scratch_types=[
            pltpu.VMEM((128,), jnp.int32),
            pltpu.VMEM((128,), jnp.int32),
            pltpu.VMEM((128,), f32),
            pltpu.VMEM((128,), f32),
            pltpu.VMEM((stripe,), f32),
            pltpu.VMEM_SHARED((NP,), f32),
            pltpu.VMEM_SHARED((NP,), f32),
            pltpu.SemaphoreType.DMA,
        ])(functools.partial(_sc_pass1, n_real=N, epw=epw1,
                             nchunks=nch1, stripe=stripe))
    deg_ns_f, deg_all_f = pass1(src_p, dst_p, zeros1)
    deg_ns_p = deg_ns_f.reshape(2, NP)
    deg_all_p = deg_all_f.reshape(2, NP)

    nblk = NP // 128
    col = lambda a: a.reshape(NP, 1)
    k3 = pl.pallas_call(
        _k3_body,
        grid=(nblk,),
        in_specs=[
            pl.BlockSpec((128, 128), lambda i: (i, 0)),
            pl.BlockSpec((128, 384), lambda i: (0, 0)),
            pl.BlockSpec((128, 1), lambda i: (i, 0)),
            pl.BlockSpec((128, 1), lambda i: (i, 0)),
            pl.BlockSpec((128, 1), lambda i: (i, 0)),
            pl.BlockSpec((128, 1), lambda i: (i, 0)),
        ],
        out_specs=[
            pl.BlockSpec((128, 384), lambda i: (i, 0)),
            pl.BlockSpec((128, 128), lambda i: (i, 0)),
            pl.BlockSpec((128, 128), lambda i: (i, 0)),
            pl.BlockSpec((128, 1), lambda i: (i, 0)),
            pl.BlockSpec((128, 1), lambda i: (i, 0)),
            pl.BlockSpec((128, 1), lambda i: (i, 0)),
        ],
        out_shape=[
            jax.ShapeDtypeStruct((NP, 384), f32),
            jax.ShapeDtypeStruct((NP, 128), f32),
            jax.ShapeDtypeStruct((NP, 128), f32),
            jax.ShapeDtypeStruct((NP, 1), f32),
            jax.ShapeDtypeStruct((NP, 1), f32),
            jax.ShapeDtypeStruct((NP, 1), f32),
        ])
    XW, Xk_s, Xv_s, xs2_s, dinv_l, dinv_ns = k3(
        x_p, Wcat, col(deg_ns_p[0]), col(deg_ns_p[1]),
        col(deg_all_p[0]), col(deg_all_p[1]))

    k3b = pl.pallas_call(
        _k3b_body,
        grid=(1,),
        in_specs=[pl.BlockSpec((NB // 128, 128), lambda i: (0, 0))],
        out_specs=pl.BlockSpec((1, 128), lambda i: (0, 0)),
        out_shape=jax.ShapeDtypeStruct((1, 128), jnp.int32))
    ptr_row = k3b(batch_rs)
    ptr_pad = jnp.concatenate(
        [ptr_row[0, :B + 1], jnp.zeros((15,), jnp.int32)])

    xs2_flat = xs2_s.reshape(NP)
    epc1 = NPE // 16
    nchp = epc1 // 128
    half = NP // 2
    acc_rows = half + 128
    zeros2 = jnp.zeros((acc_rows // 16, 128), f32)
    pass2 = functools.partial(
        pl.kernel,
        out_type=[jax.ShapeDtypeStruct((NP, 128), f32),
                  jax.ShapeDtypeStruct((NP, 128), f32),
                  jax.ShapeDtypeStruct((NP,), f32)],
        mesh=mesh,
        scratch_types=(
            [pltpu.VMEM((128,), jnp.int32)] * 8
            + [pltpu.VMEM((128, 128), f32)] * 2
            + [pltpu.VMEM((128,), f32)] * 2
            + [pltpu.VMEM((acc_rows // 16, 128), f32),
               pltpu.VMEM((stripe,), f32),
               pltpu.VMEM_SHARED((acc_rows, 128), f32),
               pltpu.VMEM_SHARED((NP,), f32)]
            + [pltpu.SemaphoreType.DMA] * 10
        ))(functools.partial(_sc_pass2, epc=epc1, nchunks=nchp,
                             stripe1=stripe, half=half, acc_rows=acc_rows))
    accK, accV, acc_s2 = pass2(src_p, dst_p, Xk_s, Xv_s, xs2_flat,
                               zeros2, zeros1)

    row = lambda a: a.reshape(1, -1)
    bs = ((b_s1[0] * ALPHA + b_s2[0] * (1.0 - ALPHA))
          .reshape(1, 1).astype(f32))
    k45 = pl.pallas_call(
        _k45_body,
        grid=(nblk,),
        in_specs=[
            pl.BlockSpec((128, 128), lambda i: (i, 0)),
            pl.BlockSpec((128, 384), lambda i: (i, 0)),
            pl.BlockSpec((128, 128), lambda i: (i, 0)),
            pl.BlockSpec((128, 128), lambda i: (i, 0)),
            pl.BlockSpec((128, 1), lambda i: (i, 0)),
            pl.BlockSpec((128, 1), lambda i: (i, 0)),
            pl.BlockSpec((128, 1), lambda i: (i, 0)),
            pl.BlockSpec((1, 128), lambda i: (0, 0)),
            pl.BlockSpec((1, 128), lambda i: (0, 0)),
            pl.BlockSpec((1, 1), lambda i: (0, 0)),
        ],
        out_specs=[
            pl.BlockSpec((128, 128), lambda i: (i, 0)),
            pl.BlockSpec((128, 128), lambda i: (i, 0)),
            pl.BlockSpec((128, 128), lambda i: (i, 0)),
        ],
        out_shape=[
            jax.ShapeDtypeStruct((NP, 128), f32),
            jax.ShapeDtypeStruct((NP, 128), f32),
            jax.ShapeDtypeStruct((NP, 128), f32),
        ])
    Kfin, Vfin, xp = k45(x_p, XW, accK, accV, col(acc_s2), dinv_l, dinv_ns,
                         row(bk), row(bv), bs)

    k5 = pl.pallas_call(
        _k5_body,
        grid=(B,),
        in_specs=[
            pl.BlockSpec(memory_space=pltpu.SMEM),
            pl.BlockSpec((NP, 128), lambda b: (0, 0)),
            pl.BlockSpec((NP, 128), lambda b: (0, 0)),
            pl.BlockSpec((NP, 128), lambda b: (0, 0)),
            pl.BlockSpec((128, 128), lambda b: (0, 0)),
            pl.BlockSpec((1, 128), lambda b: (0, 0)),
            pl.BlockSpec((128, 128), lambda b: (0, 0)),
            pl.BlockSpec((1, 128), lambda b: (0, 0)),
            pl.BlockSpec((1, 128), lambda b: (0, 0)),
            pl.BlockSpec((1, 128), lambda b: (0, 0)),
            pl.BlockSpec((1, 128), lambda b: (0, 0)),
            pl.BlockSpec((1, 128), lambda b: (0, 0)),
            pl.BlockSpec((1, 64), lambda b: (0, 0)),
            pl.BlockSpec((1, 128), lambda b: (0, 0)),
        ],
        out_specs=pl.BlockSpec((1, 1, 128), lambda b: (b, 0, 0)),
        out_shape=jax.ShapeDtypeStruct((B, 1, 128), f32))
    out = k5(ptr_pad, xp, Kfin, Vfin, Wq, row(bq), Wo, row(bo),
             row(g0), row(be0), row(g1), row(be1),
             w_r.reshape(1, RATIO), jnp.broadcast_to(b_r, (1, 128)))
    return out.reshape(B, 128)


# speedup vs baseline: 24.4137x; 1.0049x over previous
import functools

import jax
import jax.numpy as jnp
from jax import lax
from jax.experimental import pallas as pl
from jax.experimental.pallas import tpu as pltpu
from jax.experimental.pallas import tpu_sc as plsc

NHID = 128
ALPHA = 0.5
RATIO = 64
HEADS = 4
NGRAPH = 16
CH = 512


def _sc_pass1(src_h, dst_h, zeros1_h, out_ns, out_all,
              sidx, didx, vns, vall, stage, acc_ns, acc_all, sem,
              *, n_real, epw, nchunks, stripe):
    c = lax.axis_index("c")
    s = lax.axis_index("s")
    wid = s * 2 + c
    pltpu.sync_copy(zeros1_h, stage)
    pltpu.sync_copy(stage, acc_ns.at[pl.ds(s * stripe, stripe)])
    pltpu.sync_copy(stage, acc_all.at[pl.ds(s * stripe, stripe)])
    plsc.subcore_barrier()
    base0 = wid * epw

    def body(g, carry):
        base = base0 + g * 128
        pltpu.sync_copy(src_h.at[pl.ds(base, 128)], sidx)
        pltpu.sync_copy(dst_h.at[pl.ds(base, 128)], didx)
        for j in range(8):
            sv = sidx[pl.ds(j * 16, 16)]
            dv = didx[pl.ds(j * 16, 16)]
            is_real = sv < n_real
            one = jnp.ones((16,), jnp.float32)
            zero = jnp.zeros((16,), jnp.float32)
            vall[pl.ds(j * 16, 16)] = jnp.where(is_real, one, zero)
            vns[pl.ds(j * 16, 16)] = jnp.where(
                is_real & (sv != dv), one, zero)
        pltpu.sync_copy(vns, acc_ns.at[didx], add=True)
        pltpu.sync_copy(vall, acc_all.at[didx], add=True)
        return carry

    lax.fori_loop(0, nchunks, body, 0)
    plsc.subcore_barrier()
    np_tot = 16 * stripe
    pltpu.sync_copy(acc_ns.at[pl.ds(s * stripe, stripe)], stage)
    pltpu.sync_copy(stage, out_ns.at[pl.ds(c * np_tot + s * stripe, stripe)])
    pltpu.sync_copy(acc_all.at[pl.ds(s * stripe, stripe)], stage)
    pltpu.sync_copy(stage, out_all.at[pl.ds(c * np_tot + s * stripe, stripe)])


def _sc_pass2(src_h, dst_h, zk_h, zv_h, xs2_h, zeros2_h, zeros1_h,
              outk, outv, s2_o,
              sidx0, sidx1, didx0, didx1, didx20, didx21, didx30, didx31,
              rows0, rows1, valb0, valb1, stage, stage1, acc, acc_s2,
              gsem0, gsem1, isem0, isem1, ssem0, ssem1,
              xsem0, xsem1, s2sem0, s2sem1,
              *, epc, nchunks, stripe1, half, acc_rows):
    c = lax.axis_index("c")
    s = lax.axis_index("s")
    n = nchunks
    stripe_a = acc_rows // 16
    stripe_o = half // 16
    pltpu.sync_copy(zeros1_h, stage1)
    pltpu.sync_copy(stage1, acc_s2.at[pl.ds(s * stripe1, stripe1)])
    base0 = s * epc
    slot = [
        (sidx0, didx0, didx20, didx30, rows0, valb0,
         gsem0, isem0, ssem0, xsem0, s2sem0),
        (sidx1, didx1, didx21, didx31, rows1, valb1,
         gsem1, isem1, ssem1, xsem1, s2sem1),
    ]

    def issue_gather(sidx, rows, gsem, valb, xsem, do_s2):
        @pl.when(c == 0)
        def _():
            pltpu.async_copy(zk_h.at[sidx], rows, gsem)
            if do_s2:
                pltpu.async_copy(xs2_h.at[sidx], valb, xsem)

        @pl.when(c == 1)
        def _():
            pltpu.async_copy(zv_h.at[sidx], rows, gsem)

    def wait_gather(sidx, rows, gsem, valb, xsem, do_s2):
        @pl.when(c == 0)
        def _():
            pltpu.make_async_copy(zk_h.at[sidx], rows, gsem).wait()
            if do_s2:
                pltpu.make_async_copy(xs2_h.at[sidx], valb, xsem).wait()

        @pl.when(c == 1)
        def _():
            pltpu.make_async_copy(zv_h.at[sidx], rows, gsem).wait()

    for p in range(2):
        lo = p * half
        do_s2 = (p == 0)
        pltpu.sync_copy(zeros2_h, stage)
        pltpu.sync_copy(stage, acc.at[pl.ds(s * stripe_a, stripe_a)])
        plsc.subcore_barrier()

        pltpu.sync_copy(src_h.at[pl.ds(base0, 128)], sidx0)
        pltpu.sync_copy(dst_h.at[pl.ds(base0, 128)], didx0)
        issue_gather(sidx0, rows0, gsem0, valb0, xsem0, do_s2)
        pltpu.async_copy(src_h.at[pl.ds(base0 + 128, 128)], sidx1, isem1)
        pltpu.async_copy(dst_h.at[pl.ds(base0 + 128, 128)], didx1, isem1)

        def substep(g, b):
            (sidxb, didxb, didx2b, didx3b, rowsb, valbb,
             gsemb, isemb, ssemb, xsemb, s2semb) = slot[b]
            (sidxo, didxo, didx2o, didx3o, rowso, valbo,
             gsemo, isemo, ssemo, xsemo, s2semo) = slot[1 - b]

            @pl.when(g >= 1)
            def _():
                pltpu.make_async_copy(rowso, acc.at[didx2o], ssemo).wait()
                if do_s2:
                    @pl.when(c == 0)
                    def _():
                        pltpu.make_async_copy(
                            valbo, acc_s2.at[didx3o], s2semo).wait()

            @pl.when(g + 1 < n)
            def _():
                base1 = base0 + (g + 1) * 128
                pltpu.make_async_copy(
                    src_h.at[pl.ds(base1, 128)], sidxo, isemo).wait()
                pltpu.make_async_copy(
                    dst_h.at[pl.ds(base1, 128)], didxo, isemo).wait()
                issue_gather(sidxo, rowso, gsemo, valbo, xsemo, do_s2)

            wait_gather(sidxb, rowsb, gsemb, valbb, xsemb, do_s2)
            for j in range(8):
                dv = didxb[pl.ds(j * 16, 16)]
                in_half = (dv >= lo) & (dv < lo + half)
                didx2b[pl.ds(j * 16, 16)] = jnp.where(
                    in_half, dv - lo, half + (dv & 127))
            if do_s2:
                @pl.when(c == 0)
                def _():
                    for j in range(8):
                        sv = sidxb[pl.ds(j * 16, 16)]
                        dv = didxb[pl.ds(j * 16, 16)]
                        g16 = valbb[pl.ds(j * 16, 16)]
                        valbb[pl.ds(j * 16, 16)] = jnp.where(
                            sv != dv, g16, jnp.zeros((16,), jnp.float32))
                        didx3b[pl.ds(j * 16, 16)] = dv

            @pl.when(g + 2 < n)
            def _():
                base2 = base0 + (g + 2) * 128
                pltpu.async_copy(src_h.at[pl.ds(base2, 128)], sidxb, isemb)
                pltpu.async_copy(dst_h.at[pl.ds(base2, 128)], didxb, isemb)

            pltpu.async_copy(rowsb, acc.at[didx2b], ssemb, add=True)
            if do_s2:
                @pl.when(c == 0)
                def _():
                    pltpu.async_copy(valbb, acc_s2.at[didx3b], s2semb,
                                     add=True)

        def body(t, carry):
            substep(2 * t, 0)
            substep(2 * t + 1, 1)
            return carry

        lax.fori_loop(0, n // 2, body, 0)
        pltpu.make_async_copy(rows1, acc.at[didx21], ssem1).wait()
        if do_s2:
            @pl.when(c == 0)
            def _():
                pltpu.make_async_copy(valb1, acc_s2.at[didx31],
                                      s2sem1).wait()
        plsc.subcore_barrier()
        pltpu.sync_copy(acc.at[pl.ds(s * stripe_o, stripe_o)],
                        stage.at[pl.ds(0, stripe_o)])

        @pl.when(c == 0)
        def _():
            pltpu.sync_copy(stage.at[pl.ds(0, stripe_o)],
                            outk.at[pl.ds(lo + s * stripe_o, stripe_o)])

        @pl.when(c == 1)
        def _():
            pltpu.sync_copy(stage.at[pl.ds(0, stripe_o)],
                            outv.at[pl.ds(lo + s * stripe_o, stripe_o)])

        if p == 0:
            @pl.when(c == 0)
            def _():
                pltpu.sync_copy(acc_s2.at[pl.ds(s * stripe1, stripe1)],
                                stage1)
                pltpu.sync_copy(stage1,
                                s2_o.at[pl.ds(s * stripe1, stripe1)])
        plsc.subcore_barrier()


def _k3a_body(x_ref, w_ref, batch_ref, xw_o, ptr_o):
    xw_o[...] = jnp.dot(x_ref[...], w_ref[...],
                        preferred_element_type=jnp.float32)

    @pl.when(pl.program_id(0) == 0)
    def _():
        blk = batch_ref[...]
        lane = lax.broadcasted_iota(jnp.int32, (1, 128), 1)
        row = jnp.zeros((1, 128), jnp.int32)
        for b in range(NGRAPH + 1):
            cnt = jnp.sum((blk < b).astype(jnp.int32))
            row = jnp.where(lane == b, cnt, row)
        ptr_o[...] = row


def _k3c_body(xw_ref, ns0, ns1, al0, al1,
              xk_o, xv_o, xs2_o, dl_o, dn_o, s1_o):
    xw = xw_ref[...]
    dns = ns0[...] + ns1[...]
    dal = al0[...] + al1[...]
    dinv_ns = jnp.where(dns > 0, lax.rsqrt(jnp.where(dns > 0, dns, 1.0)), 0.0)
    dinv_l = lax.rsqrt(dal + 1.0)
    xk_o[...] = dinv_l * xw[:, 0:128]
    xv_o[...] = dinv_l * xw[:, 128:256]
    xs2_o[...] = dinv_ns * xw[:, 256:257]
    dl_o[...] = dinv_l
    dn_o[...] = dinv_ns
    s1_o[...] = xw[:, 257:258]


def _k45_body(x_ref, xk_ref, xv_ref, ak_ref, av_ref, as2_ref,
              dl_ref, dn_ref, s1_ref, bk_ref, bv_ref, bs_ref,
              kf_o, vf_o, xp_o):
    dl = dl_ref[...]
    dn = dn_ref[...]
    kf_o[...] = dl * (ak_ref[...] + xk_ref[...]) + bk_ref[...]
    vf_o[...] = dl * (av_ref[...] + xv_ref[...]) + bv_ref[...]
    s2 = dn * as2_ref[...]
    score = ALPHA * s1_ref[...] + (1.0 - ALPHA) * s2 + bs_ref[0, 0]
    xp_o[...] = x_ref[...] * jnp.tanh(score)


def _k5_body(ptr_ref, xp_ref, k_ref, v_ref, wq_ref, bq_ref, wo_ref, bo_ref,
             g0_ref, be0_ref, g1_ref, be1_ref, wr_ref, br_ref, out_ref):
    b = pl.program_id(0)
    p0 = ptr_ref[b]
    cnt = ptr_ref[b + 1] - p0
    rows = xp_ref[pl.ds(p0, RATIO), :]
    r_iota = lax.broadcasted_iota(jnp.int32, (RATIO, 1), 0)
    qd = jnp.where(r_iota < cnt, rows, 0.0)
    q = jnp.dot(qd, wq_ref[...], preferred_element_type=jnp.float32) + bq_ref[...]
    scale = 1.0 / jnp.sqrt(jnp.asarray(float(NHID), jnp.float32))
    hd = NHID // HEADS
    nch = (cnt + CH - 1) // CH
    heads = []
    for h in range(HEADS):
        qh = q[:, h * hd:(h + 1) * hd]

        def body(j, carry):
            m, l, acc = carry
            base = p0 + j * CH
            kc = k_ref[pl.ds(base, CH), :][:, h * hd:(h + 1) * hd]
            vc = v_ref[pl.ds(base, CH), :][:, h * hd:(h + 1) * hd]
            sA = lax.dot_general(qh, kc, (((1,), (1,)), ((), ())),
                                 preferred_element_type=jnp.float32) * scale
            col = lax.broadcasted_iota(jnp.int32, (RATIO, CH), 1)
            sA = sA + jnp.where(col < (cnt - j * CH), 0.0, -1e9)
            m_new = jnp.maximum(m, jnp.max(sA, axis=1, keepdims=True))
            p = jnp.exp(sA - m_new)
            corr = jnp.exp(m - m_new)
            l_new = l * corr + jnp.sum(p, axis=1, keepdims=True)
            acc_new = acc * corr + jnp.dot(p, vc,
                                           preferred_element_type=jnp.float32)
            return m_new, l_new, acc_new

        m0 = jnp.full((RATIO, 1), -1e30, jnp.float32)
        l0 = jnp.zeros((RATIO, 1), jnp.float32)
        a0 = jnp.zeros((RATIO, hd), jnp.float32)
        m, l, acc = lax.fori_loop(0, nch, body, (m0, l0, a0))
        heads.append(jnp.where(l > 0, acc / jnp.where(l > 0, l, 1.0), 0.0))
    o = q + jnp.concatenate(heads, axis=1)

    def ln(t, g, be):
        mu = jnp.mean(t, axis=-1, keepdims=True)
        var = jnp.mean((t - mu) ** 2, axis=-1, keepdims=True)
        return (t - mu) * lax.rsqrt(var + 1e-5) * g + be

    o = ln(o, g0_ref[...], be0_ref[...])
    o = o + jax.nn.relu(jnp.dot(o, wo_ref[...],
                                preferred_element_type=jnp.float32) + bo_ref[...])
    o = ln(o, g1_ref[...], be1_ref[...])
    out_ref[0] = jnp.dot(wr_ref[...], o,
                         preferred_element_type=jnp.float32) + br_ref[...]


def kernel(x, edge_index, batch, w_s1, b_s1, w_s2, b_s2, Wq, bq, Wk, bk,
           Wv, bv, g0, be0, Wo, bo, g1, be1, w_r, b_r):
    N, D = x.shape
    E = edge_index.shape[1]
    B = NGRAPH
    NP = ((N + 511 + 255) // 256) * 256
    stripe = NP // 16
    NPE = ((E + 4095) // 4096) * 4096
    epw1 = NPE // 32
    nch1 = epw1 // 128
    epc2 = NPE // 16
    nch2 = epc2 // 128
    NB = ((N + 1023) // 1024) * 1024

    f32 = jnp.float32
    x_p = jnp.zeros((NP, D), f32).at[:N].set(x)
    Wcat = jnp.concatenate(
        [Wk, Wv, w_s2, w_s1, jnp.zeros((D, 126), f32)], axis=1)
    npad = NPE - E
    src_p = jnp.concatenate(
        [edge_index[0],
         N + (jnp.arange(npad, dtype=jnp.int32) % (NP - N))])
    dst_p = jnp.concatenate(
        [edge_index[1], jnp.arange(npad, dtype=jnp.int32) % N])
    batch_rs = jnp.concatenate(
        [batch, jnp.full((NB - N,), B, jnp.int32)]).reshape(NB // 128, 128)
    zeros1 = jnp.zeros((stripe,), f32)

    mesh = plsc.VectorSubcoreMesh(core_axis_name="c", subcore_axis_name="s")
    pass1 = functools.partial(
        pl.kernel,
        out_type=[jax.ShapeDtypeStruct((2 * NP,), f32),
                  jax.ShapeDtypeStruct((2 * NP,), f32)],
        mesh=mesh,
        scratch_types=[
            pltpu.VMEM((128,), jnp.int32),
            pltpu.VMEM((128,), jnp.int32),
            pltpu.VMEM((128,), f32),
            pltpu.VMEM((128,), f32),
            pltpu.VMEM((stripe,), f32),
            pltpu.VMEM_SHARED((NP,), f32),
            pltpu.VMEM_SHARED((NP,), f32),
            pltpu.SemaphoreType.DMA,
        ])(functools.partial(_sc_pass1, n_real=N, epw=epw1,
                             nchunks=nch1, stripe=stripe))
    deg_ns_f, deg_all_f = pass1(src_p, dst_p, zeros1)
    deg_ns_p = deg_ns_f.reshape(2, NP)
    deg_all_p = deg_all_f.reshape(2, NP)

    nblk = NP // 128
    col = lambda a: a.reshape(NP, 1)
    k3a = pl.pallas_call(
        _k3a_body,
        grid=(nblk,),
        in_specs=[
            pl.BlockSpec((128, 128), lambda i: (i, 0)),
            pl.BlockSpec((128, 384), lambda i: (0, 0)),
            pl.BlockSpec((NB // 128, 128), lambda i: (0, 0)),
        ],
        out_specs=[
            pl.BlockSpec((128, 384), lambda i: (i, 0)),
            pl.BlockSpec((1, 128), lambda i: (0, 0)),
        ],
        out_shape=[
            jax.ShapeDtypeStruct((NP, 384), f32),
            jax.ShapeDtypeStruct((1, 128), jnp.int32),
        ])
    XW, ptr_row = k3a(x_p, Wcat, batch_rs)
    ptr_pad = jnp.concatenate(
        [ptr_row[0, :B + 1], jnp.zeros((15,), jnp.int32)])

    k3c = pl.pallas_call(
        _k3c_body,
        grid=(nblk,),
        in_specs=[
            pl.BlockSpec((128, 384), lambda i: (i, 0)),
            pl.BlockSpec((128, 1), lambda i: (i, 0)),
            pl.BlockSpec((128, 1), lambda i: (i, 0)),
            pl.BlockSpec((128, 1), lambda i: (i, 0)),
            pl.BlockSpec((128, 1), lambda i: (i, 0)),
        ],
        out_specs=[
            pl.BlockSpec((128, 128), lambda i: (i, 0)),
            pl.BlockSpec((128, 128), lambda i: (i, 0)),
            pl.BlockSpec((128, 1), lambda i: (i, 0)),
            pl.BlockSpec((128, 1), lambda i: (i, 0)),
            pl.BlockSpec((128, 1), lambda i: (i, 0)),
            pl.BlockSpec((128, 1), lambda i: (i, 0)),
        ],
        out_shape=[
            jax.ShapeDtypeStruct((NP, 128), f32),
            jax.ShapeDtypeStruct((NP, 128), f32),
            jax.ShapeDtypeStruct((NP, 1), f32),
            jax.ShapeDtypeStruct((NP, 1), f32),
            jax.ShapeDtypeStruct((NP, 1), f32),
            jax.ShapeDtypeStruct((NP, 1), f32),
        ])
    Xk_s, Xv_s, xs2_s, dinv_l, dinv_ns, s1col = k3c(
        XW, col(deg_ns_p[0]), col(deg_ns_p[1]),
        col(deg_all_p[0]), col(deg_all_p[1]))

    xs2_flat = xs2_s.reshape(NP)
    epc1 = NPE // 16
    nchp = epc1 // 128
    half = NP // 2
    acc_rows = half + 128
    zeros2 = jnp.zeros((acc_rows // 16, 128), f32)
    pass2 = functools.partial(
        pl.kernel,
        out_type=[jax.ShapeDtypeStruct((NP, 128), f32),
                  jax.ShapeDtypeStruct((NP, 128), f32),
                  jax.ShapeDtypeStruct((NP,), f32)],
        mesh=mesh,
        scratch_types=(
            [pltpu.VMEM((128,), jnp.int32)] * 8
            + [pltpu.VMEM((128, 128), f32)] * 2
            + [pltpu.VMEM((128,), f32)] * 2
            + [pltpu.VMEM((acc_rows // 16, 128), f32),
               pltpu.VMEM((stripe,), f32),
               pltpu.VMEM_SHARED((acc_rows, 128), f32),
               pltpu.VMEM_SHARED((NP,), f32)]
            + [pltpu.SemaphoreType.DMA] * 10
        ))(functools.partial(_sc_pass2, epc=epc1, nchunks=nchp,
                             stripe1=stripe, half=half, acc_rows=acc_rows))
    accK, accV, acc_s2 = pass2(src_p, dst_p, Xk_s, Xv_s, xs2_flat,
                               zeros2, zeros1)

    row = lambda a: a.reshape(1, -1)
    bs = ((b_s1[0] * ALPHA + b_s2[0] * (1.0 - ALPHA))
          .reshape(1, 1).astype(f32))
    k45 = pl.pallas_call(
        _k45_body,
        grid=(nblk,),
        in_specs=[
            pl.BlockSpec((128, 128), lambda i: (i, 0)),
            pl.BlockSpec((128, 128), lambda i: (i, 0)),
            pl.BlockSpec((128, 128), lambda i: (i, 0)),
            pl.BlockSpec((128, 128), lambda i: (i, 0)),
            pl.BlockSpec((128, 128), lambda i: (i, 0)),
            pl.BlockSpec((128, 1), lambda i: (i, 0)),
            pl.BlockSpec((128, 1), lambda i: (i, 0)),
            pl.BlockSpec((128, 1), lambda i: (i, 0)),
            pl.BlockSpec((128, 1), lambda i: (i, 0)),
            pl.BlockSpec((1, 128), lambda i: (0, 0)),
            pl.BlockSpec((1, 128), lambda i: (0, 0)),
            pl.BlockSpec((1, 1), lambda i: (0, 0)),
        ],
        out_specs=[
            pl.BlockSpec((128, 128), lambda i: (i, 0)),
            pl.BlockSpec((128, 128), lambda i: (i, 0)),
            pl.BlockSpec((128, 128), lambda i: (i, 0)),
        ],
        out_shape=[
            jax.ShapeDtypeStruct((NP, 128), f32),
            jax.ShapeDtypeStruct((NP, 128), f32),
            jax.ShapeDtypeStruct((NP, 128), f32),
        ])
    Kfin, Vfin, xp = k45(x_p, Xk_s, Xv_s, accK, accV, col(acc_s2),
                         dinv_l, dinv_ns, s1col, row(bk), row(bv), bs)

    k5 = pl.pallas_call(
        _k5_body,
        grid=(B,),
        in_specs=[
            pl.BlockSpec(memory_space=pltpu.SMEM),
            pl.BlockSpec((NP, 128), lambda b: (0, 0)),
            pl.BlockSpec((NP, 128), lambda b: (0, 0)),
            pl.BlockSpec((NP, 128), lambda b: (0, 0)),
            pl.BlockSpec((128, 128), lambda b: (0, 0)),
            pl.BlockSpec((1, 128), lambda b: (0, 0)),
            pl.BlockSpec((128, 128), lambda b: (0, 0)),
            pl.BlockSpec((1, 128), lambda b: (0, 0)),
            pl.BlockSpec((1, 128), lambda b: (0, 0)),
            pl.BlockSpec((1, 128), lambda b: (0, 0)),
            pl.BlockSpec((1, 128), lambda b: (0, 0)),
            pl.BlockSpec((1, 128), lambda b: (0, 0)),
            pl.BlockSpec((1, 64), lambda b: (0, 0)),
            pl.BlockSpec((1, 128), lambda b: (0, 0)),
        ],
        out_specs=pl.BlockSpec((1, 1, 128), lambda b: (b, 0, 0)),
        out_shape=jax.ShapeDtypeStruct((B, 1, 128), f32))
    out = k5(ptr_pad, xp, Kfin, Vfin, Wq, row(bq), Wo, row(bo),
             row(g0), row(be0), row(g1), row(be1),
             w_r.reshape(1, RATIO), jnp.broadcast_to(b_r, (1, 128)))
    return out.reshape(B, 128)


# speedup vs baseline: 25.6111x; 1.0490x over previous
import functools

import jax
import jax.numpy as jnp
from jax import lax
from jax.experimental import pallas as pl
from jax.experimental.pallas import tpu as pltpu
from jax.experimental.pallas import tpu_sc as plsc

NHID = 128
ALPHA = 0.5
RATIO = 64
HEADS = 4
NGRAPH = 16
CH = 512


def _sc_pass1(src_h, dst_h, zeros1_h, out_ns, out_all,
              sidx0, sidx1, didx0, didx1, didxs0, didxs1,
              vns0, vns1, vall0, vall1, stage, acc_ns, acc_all,
              isem0, isem1, nsem0, nsem1, asem0, asem1,
              *, n_real, epw, nchunks, stripe):
    c = lax.axis_index("c")
    s = lax.axis_index("s")
    n = nchunks
    wid = s * 2 + c
    pltpu.sync_copy(zeros1_h, stage)
    pltpu.sync_copy(stage, acc_ns.at[pl.ds(s * stripe, stripe)])
    pltpu.sync_copy(stage, acc_all.at[pl.ds(s * stripe, stripe)])
    plsc.subcore_barrier()
    base0 = wid * epw
    slot = [
        (sidx0, didx0, didxs0, vns0, vall0, isem0, nsem0, asem0),
        (sidx1, didx1, didxs1, vns1, vall1, isem1, nsem1, asem1),
    ]
    pltpu.async_copy(src_h.at[pl.ds(base0, 128)], sidx0, isem0)
    pltpu.async_copy(dst_h.at[pl.ds(base0, 128)], didx0, isem0)
    pltpu.async_copy(src_h.at[pl.ds(base0 + 128, 128)], sidx1, isem1)
    pltpu.async_copy(dst_h.at[pl.ds(base0 + 128, 128)], didx1, isem1)

    def substep(g, b):
        sidxb, didxb, didxsb, vnsb, vallb, isemb, nsemb, asemb = slot[b]
        sidxo, didxo, didxso, vnso, vallo, isemo, nsemo, asemo = slot[1 - b]

        @pl.when(g >= 1)
        def _():
            pltpu.make_async_copy(vnso, acc_ns.at[didxso], nsemo).wait()
            pltpu.make_async_copy(vallo, acc_all.at[didxso], asemo).wait()

        base = base0 + g * 128
        pltpu.make_async_copy(src_h.at[pl.ds(base, 128)], sidxb,
                              isemb).wait()
        pltpu.make_async_copy(dst_h.at[pl.ds(base, 128)], didxb,
                              isemb).wait()
        for j in range(8):
            sv = sidxb[pl.ds(j * 16, 16)]
            dv = didxb[pl.ds(j * 16, 16)]
            is_real = sv < n_real
            one = jnp.ones((16,), jnp.float32)
            zero = jnp.zeros((16,), jnp.float32)
            vallb[pl.ds(j * 16, 16)] = jnp.where(is_real, one, zero)
            vnsb[pl.ds(j * 16, 16)] = jnp.where(
                is_real & (sv != dv), one, zero)
            didxsb[pl.ds(j * 16, 16)] = dv

        @pl.when(g + 2 < n)
        def _():
            base2 = base0 + (g + 2) * 128
            pltpu.async_copy(src_h.at[pl.ds(base2, 128)], sidxb, isemb)
            pltpu.async_copy(dst_h.at[pl.ds(base2, 128)], didxb, isemb)

        pltpu.async_copy(vnsb, acc_ns.at[didxsb], nsemb, add=True)
        pltpu.async_copy(vallb, acc_all.at[didxsb], asemb, add=True)

    def body(t, carry):
        substep(2 * t, 0)
        substep(2 * t + 1, 1)
        return carry

    lax.fori_loop(0, n // 2, body, 0)
    pltpu.make_async_copy(vns1, acc_ns.at[didxs1], nsem1).wait()
    pltpu.make_async_copy(vall1, acc_all.at[didxs1], asem1).wait()
    plsc.subcore_barrier()
    np_tot = 16 * stripe
    pltpu.sync_copy(acc_ns.at[pl.ds(s * stripe, stripe)], stage)
    pltpu.sync_copy(stage, out_ns.at[pl.ds(c * np_tot + s * stripe, stripe)])
    pltpu.sync_copy(acc_all.at[pl.ds(s * stripe, stripe)], stage)
    pltpu.sync_copy(stage, out_all.at[pl.ds(c * np_tot + s * stripe, stripe)])


def _sc_pass2(src_h, dst_h, zk_h, zv_h, xs2_h, zeros2_h, zeros1_h,
              outk, outv, s2_o,
              sidx0, sidx1, didx0, didx1, didx20, didx21, didx30, didx31,
              rows0, rows1, valb0, valb1, stage, stage1, acc, acc_s2,
              gsem0, gsem1, isem0, isem1, ssem0, ssem1,
              xsem0, xsem1, s2sem0, s2sem1,
              *, epc, nchunks, stripe1, half, acc_rows):
    c = lax.axis_index("c")
    s = lax.axis_index("s")
    n = nchunks
    stripe_a = acc_rows // 16
    stripe_o = half // 16
    pltpu.sync_copy(zeros1_h, stage1)
    pltpu.sync_copy(stage1, acc_s2.at[pl.ds(s * stripe1, stripe1)])
    base0 = s * epc
    slot = [
        (sidx0, didx0, didx20, didx30, rows0, valb0,
         gsem0, isem0, ssem0, xsem0, s2sem0),
        (sidx1, didx1, didx21, didx31, rows1, valb1,
         gsem1, isem1, ssem1, xsem1, s2sem1),
    ]

    def issue_gather(sidx, rows, gsem, valb, xsem, do_s2):
        @pl.when(c == 0)
        def _():
            pltpu.async_copy(zk_h.at[sidx], rows, gsem)
            if do_s2:
                pltpu.async_copy(xs2_h.at[sidx], valb, xsem)

        @pl.when(c == 1)
        def _():
            pltpu.async_copy(zv_h.at[sidx], rows, gsem)

    def wait_gather(sidx, rows, gsem, valb, xsem, do_s2):
        @pl.when(c == 0)
        def _():
            pltpu.make_async_copy(zk_h.at[sidx], rows, gsem).wait()
            if do_s2:
                pltpu.make_async_copy(xs2_h.at[sidx], valb, xsem).wait()

        @pl.when(c == 1)
        def _():
            pltpu.make_async_copy(zv_h.at[sidx], rows, gsem).wait()

    for p in range(2):
        lo = p * half
        do_s2 = (p == 0)
        pltpu.sync_copy(zeros2_h, stage)
        pltpu.sync_copy(stage, acc.at[pl.ds(s * stripe_a, stripe_a)])
        plsc.subcore_barrier()

        pltpu.sync_copy(src_h.at[pl.ds(base0, 128)], sidx0)
        pltpu.sync_copy(dst_h.at[pl.ds(base0, 128)], didx0)
        issue_gather(sidx0, rows0, gsem0, valb0, xsem0, do_s2)
        pltpu.async_copy(src_h.at[pl.ds(base0 + 128, 128)], sidx1, isem1)
        pltpu.async_copy(dst_h.at[pl.ds(base0 + 128, 128)], didx1, isem1)

        def substep(g, b):
            (sidxb, didxb, didx2b, didx3b, rowsb, valbb,
             gsemb, isemb, ssemb, xsemb, s2semb) = slot[b]
            (sidxo, didxo, didx2o, didx3o, rowso, valbo,
             gsemo, isemo, ssemo, xsemo, s2semo) = slot[1 - b]

            @pl.when(g >= 1)
            def _():
                pltpu.make_async_copy(rowso, acc.at[didx2o], ssemo).wait()
                if do_s2:
                    @pl.when(c == 0)
                    def _():
                        pltpu.make_async_copy(
                            valbo, acc_s2.at[didx3o], s2semo).wait()

            @pl.when(g + 1 < n)
            def _():
                base1 = base0 + (g + 1) * 128
                pltpu.make_async_copy(
                    src_h.at[pl.ds(base1, 128)], sidxo, isemo).wait()
                pltpu.make_async_copy(
                    dst_h.at[pl.ds(base1, 128)], didxo, isemo).wait()
                issue_gather(sidxo, rowso, gsemo, valbo, xsemo, do_s2)

            wait_gather(sidxb, rowsb, gsemb, valbb, xsemb, do_s2)
            for j in range(8):
                dv = didxb[pl.ds(j * 16, 16)]
                in_half = (dv >= lo) & (dv < lo + half)
                didx2b[pl.ds(j * 16, 16)] = jnp.where(
                    in_half, dv - lo, half + (dv & 127))
            if do_s2:
                @pl.when(c == 0)
                def _():
                    for j in range(8):
                        sv = sidxb[pl.ds(j * 16, 16)]
                        dv = didxb[pl.ds(j * 16, 16)]
                        g16 = valbb[pl.ds(j * 16, 16)]
                        valbb[pl.ds(j * 16, 16)] = jnp.where(
                            sv != dv, g16, jnp.zeros((16,), jnp.float32))
                        didx3b[pl.ds(j * 16, 16)] = dv

            @pl.when(g + 2 < n)
            def _():
                base2 = base0 + (g + 2) * 128
                pltpu.async_copy(src_h.at[pl.ds(base2, 128)], sidxb, isemb)
                pltpu.async_copy(dst_h.at[pl.ds(base2, 128)], didxb, isemb)

            pltpu.async_copy(rowsb, acc.at[didx2b], ssemb, add=True)
            if do_s2:
                @pl.when(c == 0)
                def _():
                    pltpu.async_copy(valbb, acc_s2.at[didx3b], s2semb,
                                     add=True)

        def body(t, carry):
            substep(2 * t, 0)
            substep(2 * t + 1, 1)
            return carry

        lax.fori_loop(0, n // 2, body, 0)
        pltpu.make_async_copy(rows1, acc.at[didx21], ssem1).wait()
        if do_s2:
            @pl.when(c == 0)
            def _():
                pltpu.make_async_copy(valb1, acc_s2.at[didx31],
                                      s2sem1).wait()
        plsc.subcore_barrier()
        pltpu.sync_copy(acc.at[pl.ds(s * stripe_o, stripe_o)],
                        stage.at[pl.ds(0, stripe_o)])

        @pl.when(c == 0)
        def _():
            pltpu.sync_copy(stage.at[pl.ds(0, stripe_o)],
                            outk.at[pl.ds(lo + s * stripe_o, stripe_o)])

        @pl.when(c == 1)
        def _():
            pltpu.sync_copy(stage.at[pl.ds(0, stripe_o)],
                            outv.at[pl.ds(lo + s * stripe_o, stripe_o)])

        if p == 0:
            @pl.when(c == 0)
            def _():
                pltpu.sync_copy(acc_s2.at[pl.ds(s * stripe1, stripe1)],
                                stage1)
                pltpu.sync_copy(stage1,
                                s2_o.at[pl.ds(s * stripe1, stripe1)])
        plsc.subcore_barrier()


def _k3a_body(x_ref, w_ref, batch_ref, xw_o, ptr_o):
    xw_o[...] = jnp.dot(x_ref[...], w_ref[...],
                        preferred_element_type=jnp.float32)

    @pl.when(pl.program_id(0) == 0)
    def _():
        blk = batch_ref[...]
        lane = lax.broadcasted_iota(jnp.int32, (1, 128), 1)
        row = jnp.zeros((1, 128), jnp.int32)
        for b in range(NGRAPH + 1):
            cnt = jnp.sum((blk < b).astype(jnp.int32))
            row = jnp.where(lane == b, cnt, row)
        ptr_o[...] = row


def _k3c_body(xw_ref, ns0, ns1, al0, al1,
              xk_o, xv_o, xs2_o, dl_o, dn_o, s1_o):
    xw = xw_ref[...]
    dns = ns0[...] + ns1[...]
    dal = al0[...] + al1[...]
    dinv_ns = jnp.where(dns > 0, lax.rsqrt(jnp.where(dns > 0, dns, 1.0)), 0.0)
    dinv_l = lax.rsqrt(dal + 1.0)
    xk_o[...] = dinv_l * xw[:, 0:128]
    xv_o[...] = dinv_l * xw[:, 128:256]
    xs2_o[...] = dinv_ns * xw[:, 256:257]
    dl_o[...] = dinv_l
    dn_o[...] = dinv_ns
    s1_o[...] = xw[:, 257:258]


def _k45_body(x_ref, xk_ref, xv_ref, ak_ref, av_ref, as2_ref,
              dl_ref, dn_ref, s1_ref, bk_ref, bv_ref, bs_ref,
              kf_o, vf_o, xp_o):
    dl = dl_ref[...]
    dn = dn_ref[...]
    kf_o[...] = dl * (ak_ref[...] + xk_ref[...]) + bk_ref[...]
    vf_o[...] = dl * (av_ref[...] + xv_ref[...]) + bv_ref[...]
    s2 = dn * as2_ref[...]
    score = ALPHA * s1_ref[...] + (1.0 - ALPHA) * s2 + bs_ref[0, 0]
    xp_o[...] = x_ref[...] * jnp.tanh(score)


def _k5_body(ptr_ref, xp_ref, k_ref, v_ref, wq_ref, bq_ref, wo_ref, bo_ref,
             g0_ref, be0_ref, g1_ref, be1_ref, wr_ref, br_ref, out_ref):
    b = pl.program_id(0)
    p0 = ptr_ref[b]
    cnt = ptr_ref[b + 1] - p0
    rows = xp_ref[pl.ds(p0, RATIO), :]
    r_iota = lax.broadcasted_iota(jnp.int32, (RATIO, 1), 0)
    qd = jnp.where(r_iota < cnt, rows, 0.0)
    q = jnp.dot(qd, wq_ref[...], preferred_element_type=jnp.float32) + bq_ref[...]
    scale = 1.0 / jnp.sqrt(jnp.asarray(float(NHID), jnp.float32))
    hd = NHID // HEADS
    nch = (cnt + CH - 1) // CH
    heads = []
    for h in range(HEADS):
        qh = q[:, h * hd:(h + 1) * hd]

        def body(j, carry):
            m, l, acc = carry
            base = p0 + j * CH
            kc = k_ref[pl.ds(base, CH), :][:, h * hd:(h + 1) * hd]
            vc = v_ref[pl.ds(base, CH), :][:, h * hd:(h + 1) * hd]
            sA = lax.dot_general(qh, kc, (((1,), (1,)), ((), ())),
                                 preferred_element_type=jnp.float32) * scale
            col = lax.broadcasted_iota(jnp.int32, (RATIO, CH), 1)
            sA = sA + jnp.where(col < (cnt - j * CH), 0.0, -1e9)
            m_new = jnp.maximum(m, jnp.max(sA, axis=1, keepdims=True))
            p = jnp.exp(sA - m_new)
            corr = jnp.exp(m - m_new)
            l_new = l * corr + jnp.sum(p, axis=1, keepdims=True)
            acc_new = acc * corr + jnp.dot(p, vc,
                                           preferred_element_type=jnp.float32)
            return m_new, l_new, acc_new

        m0 = jnp.full((RATIO, 1), -1e30, jnp.float32)
        l0 = jnp.zeros((RATIO, 1), jnp.float32)
        a0 = jnp.zeros((RATIO, hd), jnp.float32)
        m, l, acc = lax.fori_loop(0, nch, body, (m0, l0, a0))
        heads.append(jnp.where(l > 0, acc / jnp.where(l > 0, l, 1.0), 0.0))
    o = q + jnp.concatenate(heads, axis=1)

    def ln(t, g, be):
        mu = jnp.mean(t, axis=-1, keepdims=True)
        var = jnp.mean((t - mu) ** 2, axis=-1, keepdims=True)
        return (t - mu) * lax.rsqrt(var + 1e-5) * g + be

    o = ln(o, g0_ref[...], be0_ref[...])
    o = o + jax.nn.relu(jnp.dot(o, wo_ref[...],
                                preferred_element_type=jnp.float32) + bo_ref[...])
    o = ln(o, g1_ref[...], be1_ref[...])
    out_ref[0] = jnp.dot(wr_ref[...], o,
                         preferred_element_type=jnp.float32) + br_ref[...]


def kernel(x, edge_index, batch, w_s1, b_s1, w_s2, b_s2, Wq, bq, Wk, bk,
           Wv, bv, g0, be0, Wo, bo, g1, be1, w_r, b_r):
    N, D = x.shape
    E = edge_index.shape[1]
    B = NGRAPH
    NP = ((N + 511 + 255) // 256) * 256
    stripe = NP // 16
    NPE = ((E + 8191) // 8192) * 8192
    epw1 = NPE // 32
    nch1 = epw1 // 128
    epc2 = NPE // 16
    nch2 = epc2 // 128
    NB = ((N + 1023) // 1024) * 1024

    f32 = jnp.float32
    x_p = jnp.zeros((NP, D), f32).at[:N].set(x)
    Wcat = jnp.concatenate(
        [Wk, Wv, w_s2, w_s1, jnp.zeros((D, 126), f32)], axis=1)
    npad = NPE - E
    src_p = jnp.concatenate(
        [edge_index[0],
         N + (jnp.arange(npad, dtype=jnp.int32) % (NP - N))])
    dst_p = jnp.concatenate(
        [edge_index[1], jnp.arange(npad, dtype=jnp.int32) % N])
    batch_rs = jnp.concatenate(
        [batch, jnp.full((NB - N,), B, jnp.int32)]).reshape(NB // 128, 128)
    zeros1 = jnp.zeros((stripe,), f32)

    mesh = plsc.VectorSubcoreMesh(core_axis_name="c", subcore_axis_name="s")
    pass1 = functools.partial(
        pl.kernel,
        out_type=[jax.ShapeDtypeStruct((2 * NP,), f32),
                  jax.ShapeDtypeStruct((2 * NP,), f32)],
        mesh=mesh,
        scratch_types=(
            [pltpu.VMEM((128,), jnp.int32)] * 6
            + [pltpu.VMEM((128,), f32)] * 4
            + [pltpu.VMEM((stripe,), f32),
               pltpu.VMEM_SHARED((NP,), f32),
               pltpu.VMEM_SHARED((NP,), f32)]
            + [pltpu.SemaphoreType.DMA] * 6
        ))(functools.partial(_sc_pass1, n_real=N, epw=epw1,
                             nchunks=nch1, stripe=stripe))
    deg_ns_f, deg_all_f = pass1(src_p, dst_p, zeros1)
    deg_ns_p = deg_ns_f.reshape(2, NP)
    deg_all_p = deg_all_f.reshape(2, NP)

    nblk = NP // 128
    col = lambda a: a.reshape(NP, 1)
    k3a = pl.pallas_call(
        _k3a_body,
        grid=(nblk,),
        in_specs=[
            pl.BlockSpec((128, 128), lambda i: (i, 0)),
            pl.BlockSpec((128, 384), lambda i: (0, 0)),
            pl.BlockSpec((NB // 128, 128), lambda i: (0, 0)),
        ],
        out_specs=[
            pl.BlockSpec((128, 384), lambda i: (i, 0)),
            pl.BlockSpec((1, 128), lambda i: (0, 0)),
        ],
        out_shape=[
            jax.ShapeDtypeStruct((NP, 384), f32),
            jax.ShapeDtypeStruct((1, 128), jnp.int32),
        ])
    XW, ptr_row = k3a(x_p, Wcat, batch_rs)
    ptr_pad = jnp.concatenate(
        [ptr_row[0, :B + 1], jnp.zeros((15,), jnp.int32)])

    k3c = pl.pallas_call(
        _k3c_body,
        grid=(nblk,),
        in_specs=[
            pl.BlockSpec((128, 384), lambda i: (i, 0)),
            pl.BlockSpec((128, 1), lambda i: (i, 0)),
            pl.BlockSpec((128, 1), lambda i: (i, 0)),
            pl.BlockSpec((128, 1), lambda i: (i, 0)),
            pl.BlockSpec((128, 1), lambda i: (i, 0)),
        ],
        out_specs=[
            pl.BlockSpec((128, 128), lambda i: (i, 0)),
            pl.BlockSpec((128, 128), lambda i: (i, 0)),
            pl.BlockSpec((128, 1), lambda i: (i, 0)),
            pl.BlockSpec((128, 1), lambda i: (i, 0)),
            pl.BlockSpec((128, 1), lambda i: (i, 0)),
            pl.BlockSpec((128, 1), lambda i: (i, 0)),
        ],
        out_shape=[
            jax.ShapeDtypeStruct((NP, 128), f32),
            jax.ShapeDtypeStruct((NP, 128), f32),
            jax.ShapeDtypeStruct((NP, 1), f32),
            jax.ShapeDtypeStruct((NP, 1), f32),
            jax.ShapeDtypeStruct((NP, 1), f32),
            jax.ShapeDtypeStruct((NP, 1), f32),
        ])
    Xk_s, Xv_s, xs2_s, dinv_l, dinv_ns, s1col = k3c(
        XW, col(deg_ns_p[0]), col(deg_ns_p[1]),
        col(deg_all_p[0]), col(deg_all_p[1]))

    xs2_flat = xs2_s.reshape(NP)
    epc1 = NPE // 16
    nchp = epc1 // 128
    half = NP // 2
    acc_rows = half + 128
    zeros2 = jnp.zeros((acc_rows // 16, 128), f32)
    pass2 = functools.partial(
        pl.kernel,
        out_type=[jax.ShapeDtypeStruct((NP, 128), f32),
                  jax.ShapeDtypeStruct((NP, 128), f32),
                  jax.ShapeDtypeStruct((NP,), f32)],
        mesh=mesh,
        scratch_types=(
            [pltpu.VMEM((128,), jnp.int32)] * 8
            + [pltpu.VMEM((128, 128), f32)] * 2
            + [pltpu.VMEM((128,), f32)] * 2
            + [pltpu.VMEM((acc_rows // 16, 128), f32),
               pltpu.VMEM((stripe,), f32),
               pltpu.VMEM_SHARED((acc_rows, 128), f32),
               pltpu.VMEM_SHARED((NP,), f32)]
            + [pltpu.SemaphoreType.DMA] * 10
        ))(functools.partial(_sc_pass2, epc=epc1, nchunks=nchp,
                             stripe1=stripe, half=half, acc_rows=acc_rows))
    accK, accV, acc_s2 = pass2(src_p, dst_p, Xk_s, Xv_s, xs2_flat,
                               zeros2, zeros1)

    row = lambda a: a.reshape(1, -1)
    bs = ((b_s1[0] * ALPHA + b_s2[0] * (1.0 - ALPHA))
          .reshape(1, 1).astype(f32))
    k45 = pl.pallas_call(
        _k45_body,
        grid=(nblk,),
        in_specs=[
            pl.BlockSpec((128, 128), lambda i: (i, 0)),
            pl.BlockSpec((128, 128), lambda i: (i, 0)),
            pl.BlockSpec((128, 128), lambda i: (i, 0)),
            pl.BlockSpec((128, 128), lambda i: (i, 0)),
            pl.BlockSpec((128, 128), lambda i: (i, 0)),
            pl.BlockSpec((128, 1), lambda i: (i, 0)),
            pl.BlockSpec((128, 1), lambda i: (i, 0)),
            pl.BlockSpec((128, 1), lambda i: (i, 0)),
            pl.BlockSpec((128, 1), lambda i: (i, 0)),
            pl.BlockSpec((1, 128), lambda i: (0, 0)),
            pl.BlockSpec((1, 128), lambda i: (0, 0)),
            pl.BlockSpec((1, 1), lambda i: (0, 0)),
        ],
        out_specs=[
            pl.BlockSpec((128, 128), lambda i: (i, 0)),
            pl.BlockSpec((128, 128), lambda i: (i, 0)),
            pl.BlockSpec((128, 128), lambda i: (i, 0)),
        ],
        out_shape=[
            jax.ShapeDtypeStruct((NP, 128), f32),
            jax.ShapeDtypeStruct((NP, 128), f32),
            jax.ShapeDtypeStruct((NP, 128), f32),
        ])
    Kfin, Vfin, xp = k45(x_p, Xk_s, Xv_s, accK, accV, col(acc_s2),
                         dinv_l, dinv_ns, s1col, row(bk), row(bv), bs)

    k5 = pl.pallas_call(
        _k5_body,
        grid=(B,),
        in_specs=[
            pl.BlockSpec(memory_space=pltpu.SMEM),
            pl.BlockSpec((NP, 128), lambda b: (0, 0)),
            pl.BlockSpec((NP, 128), lambda b: (0, 0)),
            pl.BlockSpec((NP, 128), lambda b: (0, 0)),
            pl.BlockSpec((128, 128), lambda b: (0, 0)),
            pl.BlockSpec((1, 128), lambda b: (0, 0)),
            pl.BlockSpec((128, 128), lambda b: (0, 0)),
            pl.BlockSpec((1, 128), lambda b: (0, 0)),
            pl.BlockSpec((1, 128), lambda b: (0, 0)),
            pl.BlockSpec((1, 128), lambda b: (0, 0)),
            pl.BlockSpec((1, 128), lambda b: (0, 0)),
            pl.BlockSpec((1, 128), lambda b: (0, 0)),
            pl.BlockSpec((1, 64), lambda b: (0, 0)),
            pl.BlockSpec((1, 128), lambda b: (0, 0)),
        ],
        out_specs=pl.BlockSpec((1, 1, 128), lambda b: (b, 0, 0)),
        out_shape=jax.ShapeDtypeStruct((B, 1, 128), f32))
    out = k5(ptr_pad, xp, Kfin, Vfin, Wq, row(bq), Wo, row(bo),
             row(g0), row(be0), row(g1), row(be1),
             w_r.reshape(1, RATIO), jnp.broadcast_to(b_r, (1, 128)))
    return out.reshape(B, 128)
